# baseline (speedup 1.0000x reference)
import functools

import jax
import jax.numpy as jnp
from jax import lax
from jax.experimental import pallas as pl
from jax.experimental.pallas import tpu as pltpu
from jax.experimental.pallas import tpu_sc as plsc

_EMBED = 128
_HIDDEN = 256
_GATHER_WINDOW = 128
_TOKEN_BLOCK = 8192
_NUMF = 4


def _sc_gather(table, flat_ids):
    n = flat_ids.shape[0]
    idx = flat_ids.reshape(1, n)
    mesh = plsc.VectorSubcoreMesh(core_axis_name="core",
                                  subcore_axis_name="subcore")

    @pl.kernel(out_type=jax.ShapeDtypeStruct((n, _EMBED), table.dtype),
               mesh=mesh)
    def gather_kernel(tab_hbm, i_hbm, o_hbm):
        def body(i_vmem, o_vmem):
            pltpu.sync_copy(tab_hbm.at[i_vmem.at[0]], o_vmem)

        pltpu.emit_pipeline(
            body,
            grid=(n // _GATHER_WINDOW,),
            in_specs=[pl.BlockSpec((1, _GATHER_WINDOW), lambda i: (0, i))],
            out_specs=[pl.BlockSpec((_GATHER_WINDOW, _EMBED),
                                    lambda i: (i, 0))],
            core_axis_name=("core", "subcore"),
            dimension_semantics=(pltpu.PARALLEL,),
        )(i_hbm, o_hbm)

    return gather_kernel(table, idx)


def _mlp_block_kernel(seqs_per_blk, emb_ref, numt_ref, w1e_ref, w1n_ref,
                      b1_ref, w2_ref, b2_ref, out_ref):
    h = jnp.dot(emb_ref[...].astype(jnp.bfloat16), w1e_ref[...],
                preferred_element_type=jnp.float32)
    nt = numt_ref[...].astype(jnp.bfloat16)
    w1n = w1n_ref[...]
    dn = (((0,), (0,)), ((), ()))
    hn = jnp.concatenate(
        [lax.dot_general(nt[_NUMF * j:_NUMF * (j + 1)], w1n, dn,
                         preferred_element_type=jnp.float32)
         for j in range(seqs_per_blk)],
        axis=0)
    h = jnp.maximum(h + hn + b1_ref[...], 0.0)
    o = jnp.dot(h.astype(jnp.bfloat16), w2_ref[...],
                preferred_element_type=jnp.float32)
    out_ref[...] = jnp.maximum(o + b2_ref[...], 0.0)


def _tc_mlp(emb, numt, w1e, w1n, b1, w2, b2):
    n = emb.shape[0]
    t = _TOKEN_BLOCK
    batch = numt.shape[1]
    seqs_per_blk = t // batch
    return pl.pallas_call(
        functools.partial(_mlp_block_kernel, seqs_per_blk),
        grid=(n // t,),
        in_specs=[
            pl.BlockSpec((t, _EMBED), lambda i: (i, 0)),
            pl.BlockSpec((_NUMF * seqs_per_blk, batch), lambda i: (i, 0)),
            pl.BlockSpec((_EMBED, _HIDDEN), lambda i: (0, 0)),
            pl.BlockSpec((_NUMF, _HIDDEN), lambda i: (0, 0)),
            pl.BlockSpec((1, _HIDDEN), lambda i: (0, 0)),
            pl.BlockSpec((_HIDDEN, _HIDDEN), lambda i: (0, 0)),
            pl.BlockSpec((1, _HIDDEN), lambda i: (0, 0)),
        ],
        out_specs=pl.BlockSpec((t, _HIDDEN), lambda i: (i, 0)),
        out_shape=jax.ShapeDtypeStruct((n, _HIDDEN), jnp.float32),
        compiler_params=pltpu.CompilerParams(
            dimension_semantics=("parallel",)),
    )(emb, numt, w1e, w1n, b1, w2, b2)


_CHUNKS = 5


def kernel(ids, numeric, table, W1, b1, W2, b2):
    B, L = ids.shape
    n = B * L
    ids_sm = ids.T.reshape(-1)
    numt = numeric.reshape(B, L * _NUMF).T
    bf = jnp.bfloat16
    w1e = W1[:_EMBED].astype(bf)
    w1n = W1[_EMBED:].astype(bf)
    w2 = W2.astype(bf)
    b1r = b1.reshape(1, _HIDDEN)
    b2r = b2.reshape(1, _HIDDEN)
    seq_c = L // _CHUNKS
    tok_c = n // _CHUNKS
    outs = []
    for c in range(_CHUNKS):
        emb_c = _sc_gather(table, ids_sm[c * tok_c:(c + 1) * tok_c])
        numt_c = numt[c * seq_c * _NUMF:(c + 1) * seq_c * _NUMF]
        o = _tc_mlp(emb_c, numt_c, w1e, w1n, b1r, w2, b2r)
        outs.append(o.reshape(seq_c, B, _HIDDEN))
    return jnp.concatenate(outs, axis=0).transpose(1, 0, 2)


# speedup vs baseline: 1.7171x; 1.7171x over previous
import functools

import jax
import jax.numpy as jnp
from jax import lax
from jax.experimental import pallas as pl
from jax.experimental.pallas import tpu as pltpu
from jax.experimental.pallas import tpu_sc as plsc

_EMBED = 128
_HIDDEN = 256
_GATHER_WINDOW = 256
_TOKEN_BLOCK = 8192
_NUMF = 4


def _sc_gather(table, flat_ids):
    n = flat_ids.shape[0]
    idx = flat_ids.reshape(1, n)
    mesh = plsc.VectorSubcoreMesh(core_axis_name="core",
                                  subcore_axis_name="subcore")

    @pl.kernel(out_type=jax.ShapeDtypeStruct((n, _EMBED), table.dtype),
               mesh=mesh)
    def gather_kernel(tab_hbm, i_hbm, o_hbm):
        def body(i_vmem, o_vmem):
            pltpu.sync_copy(tab_hbm.at[i_vmem.at[0]], o_vmem)

        pltpu.emit_pipeline(
            body,
            grid=(n // _GATHER_WINDOW,),
            in_specs=[pl.BlockSpec((1, _GATHER_WINDOW), lambda i: (0, i))],
            out_specs=[pl.BlockSpec((_GATHER_WINDOW, _EMBED),
                                    lambda i: (i, 0))],
            core_axis_name=("core", "subcore"),
            dimension_semantics=(pltpu.PARALLEL,),
        )(i_hbm, o_hbm)

    return gather_kernel(table, idx)


def _mlp_block_kernel(seqs_per_blk, emb_ref, numt_ref, w1e_ref, w1n_ref,
                      b1_ref, w2_ref, b2_ref, out_ref):
    h = jnp.dot(emb_ref[...].astype(jnp.bfloat16), w1e_ref[...],
                preferred_element_type=jnp.float32)
    nt = numt_ref[...].astype(jnp.bfloat16)
    w1n = w1n_ref[...]
    dn = (((0,), (0,)), ((), ()))
    hn = jnp.concatenate(
        [lax.dot_general(nt[_NUMF * j:_NUMF * (j + 1)], w1n, dn,
                         preferred_element_type=jnp.float32)
         for j in range(seqs_per_blk)],
        axis=0)
    h = jnp.maximum(h + hn + b1_ref[...], 0.0)
    o = jnp.dot(h.astype(jnp.bfloat16), w2_ref[...],
                preferred_element_type=jnp.float32)
    out_ref[...] = jnp.maximum(o + b2_ref[...], 0.0)


def _tc_mlp(emb, numt, w1e, w1n, b1, w2, b2):
    n = emb.shape[0]
    t = _TOKEN_BLOCK
    batch = numt.shape[1]
    seqs_per_blk = t // batch
    return pl.pallas_call(
        functools.partial(_mlp_block_kernel, seqs_per_blk),
        grid=(n // t,),
        in_specs=[
            pl.BlockSpec((t, _EMBED), lambda i: (i, 0)),
            pl.BlockSpec((_NUMF * seqs_per_blk, batch), lambda i: (i, 0)),
            pl.BlockSpec((_EMBED, _HIDDEN), lambda i: (0, 0)),
            pl.BlockSpec((_NUMF, _HIDDEN), lambda i: (0, 0)),
            pl.BlockSpec((1, _HIDDEN), lambda i: (0, 0)),
            pl.BlockSpec((_HIDDEN, _HIDDEN), lambda i: (0, 0)),
            pl.BlockSpec((1, _HIDDEN), lambda i: (0, 0)),
        ],
        out_specs=pl.BlockSpec((t, _HIDDEN), lambda i: (i, 0)),
        out_shape=jax.ShapeDtypeStruct((n, _HIDDEN), jnp.float32),
        compiler_params=pltpu.CompilerParams(
            dimension_semantics=("parallel",)),
    )(emb, numt, w1e, w1n, b1, w2, b2)


def kernel(ids, numeric, table, W1, b1, W2, b2):
    B, L = ids.shape
    n = B * L
    emb = _sc_gather(table, ids.T.reshape(-1))
    numt = numeric.reshape(B, L * _NUMF).T
    bf = jnp.bfloat16
    out = _tc_mlp(emb, numt,
                  W1[:_EMBED].astype(bf), W1[_EMBED:].astype(bf),
                  b1.reshape(1, _HIDDEN), W2.astype(bf),
                  b2.reshape(1, _HIDDEN))
    return out.reshape(L, B, _HIDDEN).transpose(1, 0, 2)
